# Initial kernel scaffold; baseline (speedup 1.0000x reference)
import jax
import jax.numpy as jnp
from jax.experimental import pallas as pl


def kernel(x, edge_index, W_ih1, W_hh1, b_ih1, b_hh1, fc_self_W1, fc_self_b1, fc_neigh_W1, W_ih2, W_hh2, b_ih2, b_hh2, fc_self_W2, fc_self_b2, fc_neigh_W2):
    raise NotImplementedError("write your pallas kernel here")


# speedup vs baseline: 1.0170x; 1.0170x over previous
import functools

import jax
import jax.numpy as jnp
from jax import lax
from jax.experimental import pallas as pl
from jax.experimental.pallas import tpu as pltpu

H = 128
G4 = 4 * H
TILE = 512
T_CAP = 1024
T_FULL = 163840


def _lstm_body(xp_hbm, wih_ref, whh_ref, bias_ref, cnt_ref, off_ref, ns_ref,
               h_in, c_in, h_out, c_out, xbuf, sem):
    h_out[...] = h_in[...]
    c_out[...] = c_in[...]
    nsteps = ns_ref[0]

    def step(t, carry):
        cnt = cnt_ref[t]
        base = off_ref[t]
        nchunk = lax.div(cnt + TILE - 1, TILE)

        def chunk(m, carry2):
            row0 = m * TILE
            cp = pltpu.make_async_copy(
                xp_hbm.at[pl.ds(base + row0, TILE), :], xbuf, sem)
            cp.start()
            cp.wait()
            h = h_out[pl.ds(row0, TILE), :]
            c = c_out[pl.ds(row0, TILE), :]
            gates = (
                jnp.dot(xbuf[...], wih_ref[...],
                        preferred_element_type=jnp.float32)
                + jnp.dot(h, whh_ref[...], preferred_element_type=jnp.float32)
                + bias_ref[...])
            gi = jax.nn.sigmoid(gates[:, 0:H])
            gf = jax.nn.sigmoid(gates[:, H:2 * H])
            gg = jnp.tanh(gates[:, 2 * H:3 * H])
            go = jax.nn.sigmoid(gates[:, 3 * H:4 * H])
            c_new = gf * c + gi * gg
            h_new = go * jnp.tanh(c_new)
            valid = (row0 + lax.broadcasted_iota(jnp.int32, (TILE, 1), 0)) < cnt
            h_out[pl.ds(row0, TILE), :] = jnp.where(valid, h_new, h)
            c_out[pl.ds(row0, TILE), :] = jnp.where(valid, c_new, c)
            return carry2

        return lax.fori_loop(0, nchunk, chunk, carry)

    lax.fori_loop(0, nsteps, step, 0)


def _run_lstm(xp, wih_t, whh_t, bias, cnt, off, nsteps, h0, c0, *, interpret=False):
    n_pad = h0.shape[0]
    out_sd = jax.ShapeDtypeStruct((n_pad, H), jnp.float32)
    return pl.pallas_call(
        _lstm_body,
        in_specs=[
            pl.BlockSpec(memory_space=pl.ANY),
            pl.BlockSpec(memory_space=pltpu.VMEM),
            pl.BlockSpec(memory_space=pltpu.VMEM),
            pl.BlockSpec(memory_space=pltpu.VMEM),
            pl.BlockSpec(memory_space=pltpu.SMEM),
            pl.BlockSpec(memory_space=pltpu.SMEM),
            pl.BlockSpec(memory_space=pltpu.SMEM),
            pl.BlockSpec(memory_space=pltpu.VMEM),
            pl.BlockSpec(memory_space=pltpu.VMEM),
        ],
        out_specs=[pl.BlockSpec(memory_space=pltpu.VMEM)] * 2,
        out_shape=[out_sd, out_sd],
        scratch_shapes=[pltpu.VMEM((TILE, H), jnp.float32),
                        pltpu.SemaphoreType.DMA],
        interpret=interpret,
    )(xp, wih_t, whh_t, bias, cnt, off, nsteps, h0, c0)


def _fc_body(x_ref, hn_ref, ws_ref, wn_ref, b_ref, o_ref, *, act):
    y = (jnp.dot(x_ref[...], ws_ref[...], preferred_element_type=jnp.float32)
         + jnp.dot(hn_ref[...], wn_ref[...], preferred_element_type=jnp.float32)
         + b_ref[...])
    o_ref[...] = act(y)


def _run_fc(x, h_nat, ws_t, wn_t, b, act, *, interpret=False):
    n = x.shape[0]
    d = x.shape[1]
    wout = ws_t.shape[1]
    blk = n
    for cand in (2000, 1000, 500, 250, 200, 125, 100, 50, 25, 10, 8, 5, 4, 2, 1):
        if n % cand == 0:
            blk = cand
            break
    grid = (n // blk,)
    return pl.pallas_call(
        functools.partial(_fc_body, act=act),
        grid=grid,
        in_specs=[
            pl.BlockSpec((blk, d), lambda i: (i, 0)),
            pl.BlockSpec((blk, H), lambda i: (i, 0)),
            pl.BlockSpec((d, wout), lambda i: (0, 0)),
            pl.BlockSpec((H, wout), lambda i: (0, 0)),
            pl.BlockSpec((1, wout), lambda i: (0, 0)),
        ],
        out_specs=pl.BlockSpec((blk, wout), lambda i: (i, 0)),
        out_shape=jax.ShapeDtypeStruct((n, wout), jnp.float32),
        interpret=interpret,
    )(x, h_nat, ws_t, wn_t, b)


def _preprocess(dst, n, e_pad):
    e = dst.shape[0]
    deg = jnp.zeros(n, jnp.int32).at[dst].add(1)
    order = jnp.argsort(dst)
    dst_s = dst[order]
    offsets = jnp.cumsum(deg) - deg
    r = jnp.arange(e, dtype=jnp.int32) - offsets[dst_s].astype(jnp.int32)
    node_order = jnp.argsort(-deg)
    rank = jnp.zeros(n, jnp.int32).at[node_order].set(
        jnp.arange(n, dtype=jnp.int32))
    hist = jnp.zeros(T_FULL, jnp.int32).at[deg].add(1)
    cnt_full = (n - jnp.cumsum(hist)).astype(jnp.int32)
    cum_full = jnp.concatenate(
        [jnp.zeros(1, jnp.int32), jnp.cumsum(cnt_full)[:-1].astype(jnp.int32)])
    pos = cum_full[r] + rank[dst_s]
    maxdeg = jnp.max(deg)
    return order, rank, cnt_full, cum_full, pos, maxdeg


def _layer(x_in, packed_src, rank, cnt_full, cum_full, maxdeg, n_pad,
           W_ih, W_hh, b_ih, b_hh, fc_self_W, fc_self_b, fc_neigh_W, act,
           interpret=False):
    n = x_in.shape[0]
    xp = x_in[packed_src]
    wih_t = W_ih.T
    whh_t = W_hh.T
    bias = (b_ih + b_hh).reshape(1, G4)
    h0 = jnp.zeros((n_pad, H), jnp.float32)
    c0 = jnp.zeros((n_pad, H), jnp.float32)
    n_outer = (maxdeg + T_CAP - 1) // T_CAP

    def body(k, hc):
        hh, cc = hc
        cnt_k = lax.dynamic_slice(cnt_full, (k * T_CAP,), (T_CAP,))
        off_k = lax.dynamic_slice(cum_full, (k * T_CAP,), (T_CAP,))
        ns = jnp.clip(maxdeg - k * T_CAP, 0, T_CAP).reshape(1)
        hh, cc = _run_lstm(xp, wih_t, whh_t, bias, cnt_k, off_k, ns, hh, cc,
                           interpret=interpret)
        return (hh, cc)

    h_fin, _ = lax.fori_loop(0, n_outer, body, (h0, c0))
    h_nat = h_fin[rank]

    wout = fc_self_W.shape[0]
    wout_pad = max(8, wout)
    ws_t = jnp.zeros((x_in.shape[1], wout_pad), jnp.float32).at[:, :wout].set(
        fc_self_W.T)
    wn_t = jnp.zeros((H, wout_pad), jnp.float32).at[:, :wout].set(fc_neigh_W.T)
    b = jnp.zeros((1, wout_pad), jnp.float32).at[0, :wout].set(fc_self_b)
    out = _run_fc(x_in, h_nat, ws_t, wn_t, b, act, interpret=interpret)
    return out[:, :wout]


def _kernel_impl(x, edge_index, W_ih1, W_hh1, b_ih1, b_hh1, fc_self_W1,
                 fc_self_b1, fc_neigh_W1, W_ih2, W_hh2, b_ih2, b_hh2,
                 fc_self_W2, fc_self_b2, fc_neigh_W2, interpret=False):
    n = x.shape[0]
    e = edge_index.shape[1]
    src = edge_index[0]
    dst = edge_index[1]
    e_pad = ((e + 2 * TILE + 7) // 8) * 8
    n_pad = ((n + TILE - 1) // TILE) * TILE

    order, rank, cnt_full, cum_full, pos, maxdeg = _preprocess(dst, n, e_pad)
    src_s = src[order]
    packed_src = jnp.zeros(e_pad, jnp.int32).at[pos].set(src_s)

    h1 = _layer(x, packed_src, rank, cnt_full, cum_full, maxdeg, n_pad,
                W_ih1, W_hh1, b_ih1, b_hh1, fc_self_W1, fc_self_b1,
                fc_neigh_W1, jax.nn.relu, interpret=interpret)
    out = _layer(h1, packed_src, rank, cnt_full, cum_full, maxdeg, n_pad,
                 W_ih2, W_hh2, b_ih2, b_hh2, fc_self_W2, fc_self_b2,
                 fc_neigh_W2, jax.nn.sigmoid, interpret=interpret)
    return out


def kernel(x, edge_index, W_ih1, W_hh1, b_ih1, b_hh1, fc_self_W1, fc_self_b1,
           fc_neigh_W1, W_ih2, W_hh2, b_ih2, b_hh2, fc_self_W2, fc_self_b2,
           fc_neigh_W2):
    return _kernel_impl(x, edge_index, W_ih1, W_hh1, b_ih1, b_hh1, fc_self_W1,
                        fc_self_b1, fc_neigh_W1, W_ih2, W_hh2, b_ih2, b_hh2,
                        fc_self_W2, fc_self_b2, fc_neigh_W2)


# speedup vs baseline: 1.0954x; 1.0771x over previous
import functools

import jax
import jax.numpy as jnp
from jax import lax
from jax.experimental import pallas as pl
from jax.experimental.pallas import tpu as pltpu

H = 128
G4 = 4 * H
TILE = 512
T_CAP = 1024
T_FULL = 163840


def _lstm_body(xp_hbm, wih_ref, whh_ref, bias_ref, base_ref, row0_ref,
               ccnt_ref, nc_ref, h_in, c_in, h_out, c_out,
               xbuf0, xbuf1, sem0, sem1):
    h_out[...] = h_in[...]
    c_out[...] = c_in[...]
    nc = nc_ref[0]

    def dma(i, xbuf, sem):
        return pltpu.make_async_copy(
            xp_hbm.at[pl.ds(base_ref[i], TILE), :], xbuf, sem)

    def compute(i, xbuf):
        row0 = row0_ref[i]
        cnt = ccnt_ref[i]
        h = h_out[pl.ds(row0, TILE), :]
        c = c_out[pl.ds(row0, TILE), :]
        gates = (
            jnp.dot(xbuf[...], wih_ref[...],
                    preferred_element_type=jnp.float32)
            + jnp.dot(h, whh_ref[...], preferred_element_type=jnp.float32)
            + bias_ref[...])
        gi = jax.nn.sigmoid(gates[:, 0:H])
        gf = jax.nn.sigmoid(gates[:, H:2 * H])
        gg = jnp.tanh(gates[:, 2 * H:3 * H])
        go = jax.nn.sigmoid(gates[:, 3 * H:4 * H])
        c_new = gf * c + gi * gg
        h_new = go * jnp.tanh(c_new)
        valid = (row0 + lax.broadcasted_iota(jnp.int32, (TILE, 1), 0)) < cnt
        h_out[pl.ds(row0, TILE), :] = jnp.where(valid, h_new, h)
        c_out[pl.ds(row0, TILE), :] = jnp.where(valid, c_new, c)

    @pl.when(nc > 0)
    def _prologue():
        dma(0, xbuf0, sem0).start()

    def pair(j, carry):
        i0 = 2 * j
        i1 = i0 + 1

        @pl.when(i1 < nc)
        def _():
            dma(i1, xbuf1, sem1).start()

        dma(i0, xbuf0, sem0).wait()
        compute(i0, xbuf0)

        @pl.when(i1 < nc)
        def _():
            @pl.when(i1 + 1 < nc)
            def _():
                dma(i1 + 1, xbuf0, sem0).start()
            dma(i1, xbuf1, sem1).wait()
            compute(i1, xbuf1)

        return carry

    lax.fori_loop(0, lax.div(nc + 1, 2), pair, 0)


def _run_lstm(xp, wih_t, whh_t, bias, base_c, row0_c, cnt_c, nc, h0, c0,
              *, interpret=False):
    n_pad = h0.shape[0]
    out_sd = jax.ShapeDtypeStruct((n_pad, H), jnp.float32)
    return pl.pallas_call(
        _lstm_body,
        in_specs=[
            pl.BlockSpec(memory_space=pl.ANY),
            pl.BlockSpec(memory_space=pltpu.VMEM),
            pl.BlockSpec(memory_space=pltpu.VMEM),
            pl.BlockSpec(memory_space=pltpu.VMEM),
            pl.BlockSpec(memory_space=pltpu.SMEM),
            pl.BlockSpec(memory_space=pltpu.SMEM),
            pl.BlockSpec(memory_space=pltpu.SMEM),
            pl.BlockSpec(memory_space=pltpu.SMEM),
            pl.BlockSpec(memory_space=pltpu.VMEM),
            pl.BlockSpec(memory_space=pltpu.VMEM),
        ],
        out_specs=[pl.BlockSpec(memory_space=pltpu.VMEM)] * 2,
        out_shape=[out_sd, out_sd],
        scratch_shapes=[pltpu.VMEM((TILE, H), jnp.float32),
                        pltpu.VMEM((TILE, H), jnp.float32),
                        pltpu.SemaphoreType.DMA,
                        pltpu.SemaphoreType.DMA],
        interpret=interpret,
    )(xp, wih_t, whh_t, bias, base_c, row0_c, cnt_c, nc, h0, c0)


def _fc_body(x_ref, hn_ref, ws_ref, wn_ref, b_ref, o_ref, *, act):
    y = (jnp.dot(x_ref[...], ws_ref[...], preferred_element_type=jnp.float32)
         + jnp.dot(hn_ref[...], wn_ref[...], preferred_element_type=jnp.float32)
         + b_ref[...])
    o_ref[...] = act(y)


def _run_fc(x, h_nat, ws_t, wn_t, b, act, *, interpret=False):
    n = x.shape[0]
    d = x.shape[1]
    wout = ws_t.shape[1]
    blk = n
    for cand in (2000, 1000, 500, 250, 200, 125, 100, 50, 25, 10, 8, 5, 4, 2, 1):
        if n % cand == 0:
            blk = cand
            break
    grid = (n // blk,)
    return pl.pallas_call(
        functools.partial(_fc_body, act=act),
        grid=grid,
        in_specs=[
            pl.BlockSpec((blk, d), lambda i: (i, 0)),
            pl.BlockSpec((blk, H), lambda i: (i, 0)),
            pl.BlockSpec((d, wout), lambda i: (0, 0)),
            pl.BlockSpec((H, wout), lambda i: (0, 0)),
            pl.BlockSpec((1, wout), lambda i: (0, 0)),
        ],
        out_specs=pl.BlockSpec((blk, wout), lambda i: (i, 0)),
        out_shape=jax.ShapeDtypeStruct((n, wout), jnp.float32),
        interpret=interpret,
    )(x, h_nat, ws_t, wn_t, b)


def _preprocess(dst, n):
    e = dst.shape[0]
    deg = jnp.zeros(n, jnp.int32).at[dst].add(1)
    order = jnp.argsort(dst)
    dst_s = dst[order]
    offsets = jnp.cumsum(deg) - deg
    r = jnp.arange(e, dtype=jnp.int32) - offsets[dst_s].astype(jnp.int32)
    node_order = jnp.argsort(-deg)
    rank = jnp.zeros(n, jnp.int32).at[node_order].set(
        jnp.arange(n, dtype=jnp.int32))
    hist = jnp.zeros(T_FULL, jnp.int32).at[deg].add(1)
    cnt_full = (n - jnp.cumsum(hist)).astype(jnp.int32)
    cum_full = jnp.concatenate(
        [jnp.zeros(1, jnp.int32), jnp.cumsum(cnt_full)[:-1].astype(jnp.int32)])
    pos = cum_full[r] + rank[dst_s]
    maxdeg = jnp.max(deg)
    return order, rank, cnt_full, cum_full, pos, maxdeg


def _chunk_meta(cnt_k, off_k, nchunk_cap):
    nch = lax.div(cnt_k + TILE - 1, TILE)
    cumch = jnp.cumsum(nch)
    nc = cumch[-1]
    ii = jnp.arange(nchunk_cap, dtype=jnp.int32)
    s = jnp.searchsorted(cumch, ii, side='right').astype(jnp.int32)
    s_cl = jnp.minimum(s, T_CAP - 1)
    prev = jnp.where(s > 0, cumch[jnp.maximum(s - 1, 0)], 0)
    m = ii - prev
    live = ii < nc
    base_c = jnp.where(live, off_k[s_cl] + m * TILE, 0)
    row0_c = jnp.where(live, m * TILE, 0)
    cnt_c = jnp.where(live, cnt_k[s_cl], 0)
    return base_c, row0_c, cnt_c, nc.reshape(1)


def _layer(x_in, packed_src, rank, cnt_full, cum_full, maxdeg, n_pad,
           nchunk_cap, W_ih, W_hh, b_ih, b_hh, fc_self_W, fc_self_b,
           fc_neigh_W, act, interpret=False):
    xp = x_in[packed_src]
    wih_t = W_ih.T
    whh_t = W_hh.T
    bias = (b_ih + b_hh).reshape(1, G4)
    h0 = jnp.zeros((n_pad, H), jnp.float32)
    c0 = jnp.zeros((n_pad, H), jnp.float32)
    n_outer = (maxdeg + T_CAP - 1) // T_CAP

    def body(k, hc):
        hh, cc = hc
        cnt_k = lax.dynamic_slice(cnt_full, (k * T_CAP,), (T_CAP,))
        off_k = lax.dynamic_slice(cum_full, (k * T_CAP,), (T_CAP,))
        base_c, row0_c, cnt_c, nc = _chunk_meta(cnt_k, off_k, nchunk_cap)
        hh, cc = _run_lstm(xp, wih_t, whh_t, bias, base_c, row0_c, cnt_c, nc,
                           hh, cc, interpret=interpret)
        return (hh, cc)

    h_fin, _ = lax.fori_loop(0, n_outer, body, (h0, c0))
    h_nat = h_fin[rank]

    wout = fc_self_W.shape[0]
    wout_pad = max(8, wout)
    ws_t = jnp.zeros((x_in.shape[1], wout_pad), jnp.float32).at[:, :wout].set(
        fc_self_W.T)
    wn_t = jnp.zeros((H, wout_pad), jnp.float32).at[:, :wout].set(fc_neigh_W.T)
    b = jnp.zeros((1, wout_pad), jnp.float32).at[0, :wout].set(fc_self_b)
    out = _run_fc(x_in, h_nat, ws_t, wn_t, b, act, interpret=interpret)
    return out[:, :wout]


def _kernel_impl(x, edge_index, W_ih1, W_hh1, b_ih1, b_hh1, fc_self_W1,
                 fc_self_b1, fc_neigh_W1, W_ih2, W_hh2, b_ih2, b_hh2,
                 fc_self_W2, fc_self_b2, fc_neigh_W2, interpret=False):
    n = x.shape[0]
    e = edge_index.shape[1]
    src = edge_index[0]
    dst = edge_index[1]
    e_pad = ((e + 2 * TILE + 7) // 8) * 8
    n_pad = ((n + TILE - 1) // TILE) * TILE
    nchunk_cap = (e + TILE - 1) // TILE + T_CAP + 8

    order, rank, cnt_full, cum_full, pos, maxdeg = _preprocess(dst, n)
    src_s = src[order]
    packed_src = jnp.zeros(e_pad, jnp.int32).at[pos].set(src_s)

    h1 = _layer(x, packed_src, rank, cnt_full, cum_full, maxdeg, n_pad,
                nchunk_cap, W_ih1, W_hh1, b_ih1, b_hh1, fc_self_W1,
                fc_self_b1, fc_neigh_W1, jax.nn.relu, interpret=interpret)
    out = _layer(h1, packed_src, rank, cnt_full, cum_full, maxdeg, n_pad,
                 nchunk_cap, W_ih2, W_hh2, b_ih2, b_hh2, fc_self_W2,
                 fc_self_b2, fc_neigh_W2, jax.nn.sigmoid, interpret=interpret)
    return out


def kernel(x, edge_index, W_ih1, W_hh1, b_ih1, b_hh1, fc_self_W1, fc_self_b1,
           fc_neigh_W1, W_ih2, W_hh2, b_ih2, b_hh2, fc_self_W2, fc_self_b2,
           fc_neigh_W2):
    return _kernel_impl(x, edge_index, W_ih1, W_hh1, b_ih1, b_hh1, fc_self_W1,
                        fc_self_b1, fc_neigh_W1, W_ih2, W_hh2, b_ih2, b_hh2,
                        fc_self_W2, fc_self_b2, fc_neigh_W2)
